# baseline (speedup 1.0000x reference)
import functools

import jax
import jax.numpy as jnp
from jax import lax
from jax.experimental import pallas as pl
from jax.experimental.pallas import tpu as pltpu
from jax.experimental.pallas import tpu_sc as plsc

N = 50000
E = 800000
NC = 2
NS = 16

AGG_W = 128
AGG_NW = 392
EPAD = NS * AGG_NW * AGG_W
SUP = 14
NSUP = AGG_NW // SUP
PAIRS = SUP // 2

DEG_NWC = AGG_NW // NC
DEG_CH = 4
NPAD = 50048
DROWS = NPAD // NS
ROWS_PER_TILE = NPAD // NS
ZB = 640
FB = 16


def _mesh():
    return plsc.VectorSubcoreMesh(
        core_axis_name="c", subcore_axis_name="s", num_cores=NC, num_subcores=NS
    )


def _sc_degree(dst_r):
    @functools.partial(
        pl.kernel,
        out_type=jax.ShapeDtypeStruct((NC * NPAD,), jnp.float32),
        mesh=_mesh(),
        scratch_types=[
            pltpu.VMEM((DEG_CH, AGG_W), jnp.int32),
            pltpu.VMEM((AGG_W,), jnp.float32),
            pltpu.VMEM((ZB,), jnp.float32),
            pltpu.VMEM_SHARED((NPAD,), jnp.float32),
        ],
        compiler_params=pltpu.CompilerParams(use_tc_tiling_on_sc=False),
    )
    def deg_kernel(dst_ref, out_ref, idxv, ones, zb, acc):
        c = lax.axis_index("c")
        s = lax.axis_index("s")
        one16 = jnp.ones((16,), jnp.float32)
        for j in range(AGG_W // 16):
            ones[pl.ds(j * 16, 16)] = one16
        z16 = jnp.zeros((16,), jnp.float32)
        for j in range(ZB // 16):
            zb[pl.ds(j * 16, 16)] = z16
        base = s * DROWS
        nzb = DROWS // ZB
        ztail = DROWS - nzb * ZB

        def zcopy(i, _):
            pltpu.sync_copy(zb, acc.at[pl.ds(base + i * ZB, ZB)])
            return 0

        lax.fori_loop(0, nzb, zcopy, 0)
        pltpu.sync_copy(zb.at[pl.ds(0, ztail)], acc.at[pl.ds(base + nzb * ZB, ztail)])
        plsc.subcore_barrier()

        wbase = c * DEG_NWC

        def body(g, _):
            pltpu.sync_copy(dst_ref.at[s, pl.ds(wbase + g * DEG_CH, DEG_CH)], idxv)
            for k in range(DEG_CH):
                pltpu.sync_copy(ones, acc.at[idxv.at[k]], add=True)
            return 0

        lax.fori_loop(0, DEG_NWC // DEG_CH, body, 0)
        plsc.subcore_barrier()

        def wb(i, _):
            pltpu.sync_copy(acc.at[pl.ds(base + i * ZB, ZB)], zb)
            pltpu.sync_copy(zb, out_ref.at[pl.ds(c * NPAD + base + i * ZB, ZB)])
            return 0

        lax.fori_loop(0, nzb, wb, 0)
        pltpu.sync_copy(acc.at[pl.ds(base + nzb * ZB, ztail)], zb.at[pl.ds(0, ztail)])
        pltpu.sync_copy(
            zb.at[pl.ds(0, ztail)],
            out_ref.at[pl.ds(c * NPAD + base + nzb * ZB, ztail)],
        )

    return deg_kernel(dst_r)


def _sc_aggregate(hs_flat, src_r, dst_r, bpc):
    @functools.partial(
        pl.kernel,
        out_type=jax.ShapeDtypeStruct((NC * bpc, NPAD, FB), jnp.float32),
        mesh=_mesh(),
        scratch_types=[
            pltpu.VMEM((SUP, AGG_W), jnp.int32),
            pltpu.VMEM((SUP, AGG_W), jnp.int32),
            pltpu.VMEM((2, AGG_W, FB), jnp.float32),
            pltpu.VMEM((2, AGG_W, FB), jnp.float32),
            pltpu.VMEM_SHARED((NPAD, FB), jnp.float32),
            pltpu.SemaphoreType.DMA,
            pltpu.SemaphoreType.DMA,
            pltpu.SemaphoreType.DMA,
            pltpu.SemaphoreType.DMA,
        ],
        compiler_params=pltpu.CompilerParams(use_tc_tiling_on_sc=False),
    )
    def agg_kernel(
        hs_ref, src_ref, dst_ref, out_ref,
        srcS, dstS, rowsA, rowsB, acc, semA, semB, semSA, semSB,
    ):
        core = lax.axis_index("c")
        s = lax.axis_index("s")
        base = s * ROWS_PER_TILE
        nz = ROWS_PER_TILE // AGG_W
        tail = ROWS_PER_TILE - nz * AGG_W
        z16 = jnp.zeros((16,), jnp.float32)

        for b in range(bpc):
            off = (core * bpc + b) * N

            def fire(p, rows, sem):
                for k in (2 * p, 2 * p + 1):
                    pltpu.async_copy(hs_ref.at[srcS.at[k]], rows.at[k - 2 * p], sem)

            def wait_g(p, rows, sem):
                for k in (2 * p, 2 * p + 1):
                    pltpu.make_async_copy(
                        hs_ref.at[srcS.at[k]], rows.at[k - 2 * p], sem
                    ).wait()

            def scat_fire(p, rows, semS):
                for k in (2 * p, 2 * p + 1):
                    pltpu.async_copy(rows.at[k - 2 * p], acc.at[dstS.at[k]], semS, add=True)

            def scat_wait(p, rows, semS):
                for k in (2 * p, 2 * p + 1):
                    pltpu.make_async_copy(
                        rows.at[k - 2 * p], acc.at[dstS.at[k]], semS
                    ).wait()

            def zfill(i, _):
                rowsA[0, i, pl.ds(0, 16)] = z16
                return 0

            lax.fori_loop(0, AGG_W, zfill, 0)

            def zcopy(i, _):
                pltpu.sync_copy(rowsA.at[0], acc.at[pl.ds(base + i * AGG_W, AGG_W)])
                return 0

            lax.fori_loop(0, nz, zcopy, 0)
            pltpu.sync_copy(
                rowsA.at[0, pl.ds(0, tail)], acc.at[pl.ds(base + nz * AGG_W, tail)]
            )
            plsc.subcore_barrier()

            def superchunk(g, _):
                pltpu.sync_copy(src_ref.at[s, pl.ds(g * SUP, SUP)], srcS)
                pltpu.sync_copy(dst_ref.at[s, pl.ds(g * SUP, SUP)], dstS)
                for k in range(SUP):
                    for j in range(AGG_W // 16):
                        srcS[k, pl.ds(j * 16, 16)] = srcS[k, pl.ds(j * 16, 16)] + off
                def bufs(p):
                    return (rowsA, semA, semSA) if p % 2 == 0 else (rowsB, semB, semSB)

                fire(0, rowsA, semA)
                fire(1, rowsB, semB)
                for p in range(PAIRS):
                    rows, sem, semS = bufs(p)
                    wait_g(p, rows, sem)
                    scat_fire(p, rows, semS)
                    if 1 <= p and p + 1 < PAIRS:
                        rows2, sem2, semS2 = bufs(p - 1)
                        scat_wait(p - 1, rows2, semS2)
                        fire(p + 1, rows2, sem2)
                scat_wait(PAIRS - 2, bufs(PAIRS - 2)[0], bufs(PAIRS - 2)[2])
                scat_wait(PAIRS - 1, bufs(PAIRS - 1)[0], bufs(PAIRS - 1)[2])
                return 0

            lax.fori_loop(0, NSUP, superchunk, 0)

            plsc.subcore_barrier()

            for cblk in range(NC):
                blk = cblk * bpc + b

                @pl.when(core == cblk)
                def _():
                    def wb(i, _):
                        pltpu.sync_copy(acc.at[pl.ds(base + i * AGG_W, AGG_W)], rowsA.at[0])
                        pltpu.sync_copy(
                            rowsA.at[0], out_ref.at[blk, pl.ds(base + i * AGG_W, AGG_W)]
                        )
                        return 0

                    lax.fori_loop(0, nz, wb, 0)
                    pltpu.sync_copy(
                        acc.at[pl.ds(base + nz * AGG_W, tail)], rowsA.at[0, pl.ds(0, tail)]
                    )
                    pltpu.sync_copy(
                        rowsA.at[0, pl.ds(0, tail)],
                        out_ref.at[blk, pl.ds(base + nz * AGG_W, tail)],
                    )

    return agg_kernel(hs_flat, src_r, dst_r)


TN = 2000
GRID = N // TN


def _tc_layer1(xp, w1p, degp):
    def body(x_ref, w_ref, d_ref, hs_ref, dinv_ref):
        h = jnp.dot(x_ref[...], w_ref[...], preferred_element_type=jnp.float32)
        p = d_ref[...]
        dinv = lax.rsqrt(p[0] + p[1] + 1.0)
        hs = h * dinv
        for j in range(4):
            hs_ref[j] = hs[:, FB * j : FB * (j + 1)]
        dinv_ref[...] = dinv

    return pl.pallas_call(
        body,
        grid=(GRID,),
        in_specs=[
            pl.BlockSpec((TN, 64), lambda i: (i, 0)),
            pl.BlockSpec((64, 64), lambda i: (0, 0)),
            pl.BlockSpec((2, TN, 1), lambda i: (0, i, 0)),
        ],
        out_specs=[
            pl.BlockSpec((4, TN, FB), lambda i: (0, i, 0)),
            pl.BlockSpec((TN, 1), lambda i: (i, 0)),
        ],
        out_shape=[
            jax.ShapeDtypeStruct((4, N, FB), jnp.float32),
            jax.ShapeDtypeStruct((N, 1), jnp.float32),
        ],
    )(xp, w1p, degp)


def _tc_layer2(agg1, hs1, dinv, b1r, w2):
    def body(a_ref, h_ref, d_ref, b_ref, w_ref, out_ref):
        dinv = d_ref[...]
        u = (a_ref[...] + h_ref[...]) * dinv[None] + b_ref[...][:, None, :]
        t = jnp.maximum(u, 0.0)
        t64 = jnp.concatenate([t[j] for j in range(4)], axis=1)
        h2 = jnp.dot(t64, w_ref[...], preferred_element_type=jnp.float32)
        hs2 = h2 * dinv
        for j in range(8):
            out_ref[j] = hs2[:, FB * j : FB * (j + 1)]

    return pl.pallas_call(
        body,
        grid=(GRID,),
        in_specs=[
            pl.BlockSpec((4, TN, FB), lambda i: (0, i, 0)),
            pl.BlockSpec((4, TN, FB), lambda i: (0, i, 0)),
            pl.BlockSpec((TN, 1), lambda i: (i, 0)),
            pl.BlockSpec((4, FB), lambda i: (0, 0)),
            pl.BlockSpec((64, 128), lambda i: (0, 0)),
        ],
        out_specs=pl.BlockSpec((8, TN, FB), lambda i: (0, i, 0)),
        out_shape=jax.ShapeDtypeStruct((8, N, FB), jnp.float32),
    )(agg1, hs1, dinv, b1r, w2)


def _tc_final(agg2, hs2, dinv, b2r):
    def body(a_ref, h_ref, d_ref, b_ref, out_ref):
        u = (a_ref[...] + h_ref[...]) * d_ref[...][None] + b_ref[...][:, None, :]
        for j in range(8):
            out_ref[:, FB * j : FB * (j + 1)] = u[j]

    return pl.pallas_call(
        body,
        grid=(GRID,),
        in_specs=[
            pl.BlockSpec((8, TN, FB), lambda i: (0, i, 0)),
            pl.BlockSpec((8, TN, FB), lambda i: (0, i, 0)),
            pl.BlockSpec((TN, 1), lambda i: (i, 0)),
            pl.BlockSpec((8, FB), lambda i: (0, 0)),
        ],
        out_specs=pl.BlockSpec((TN, 128), lambda i: (i, 0)),
        out_shape=jax.ShapeDtypeStruct((N, 128), jnp.float32),
    )(agg2, hs2, dinv, b2r)


def kernel(x, edge_index, W1, b1, W2, b2):
    src = edge_index[0].astype(jnp.int32)
    dst = edge_index[1].astype(jnp.int32)
    npad = EPAD - E
    ar = jnp.arange(npad, dtype=jnp.int32)
    src_r = jnp.concatenate([src, (ar * 997) % N]).reshape(NS, AGG_NW, AGG_W)
    dst_r = jnp.concatenate([dst, N + (ar % (NPAD - N))]).reshape(NS, AGG_NW, AGG_W)
    degp = _sc_degree(dst_r)
    degp3 = jnp.stack([degp[:N], degp[NPAD : NPAD + N]])[:, :, None]

    xp = jnp.pad(x, ((0, 0), (0, 64 - x.shape[1])))
    w1p = jnp.pad(W1, ((0, 64 - W1.shape[0]), (0, 0)))
    hs1, dinv = _tc_layer1(xp, w1p, degp3)

    agg1 = _sc_aggregate(hs1.reshape(4 * N, FB), src_r, dst_r, bpc=2)

    hs2 = _tc_layer2(agg1, hs1, dinv, b1.reshape(4, FB), W2)

    agg2 = _sc_aggregate(hs2.reshape(8 * N, FB), src_r, dst_r, bpc=4)

    return _tc_final(agg2, hs2, dinv, b2.reshape(8, FB))


# speedup vs baseline: 1.1487x; 1.1487x over previous
import functools

import jax
import jax.numpy as jnp
from jax import lax
from jax.experimental import pallas as pl
from jax.experimental.pallas import tpu as pltpu
from jax.experimental.pallas import tpu_sc as plsc

N = 50000
E = 800000
NC = 2
NS = 16

AGG_W = 128
AGG_NW = 392
EPAD = NS * AGG_NW * AGG_W
SUP = 8
NSUP = AGG_NW // SUP

DEG_NWC = AGG_NW // NC
DEG_CH = 4
NPAD = 50048
DROWS = NPAD // NS
ROWS_PER_TILE = NPAD // NS
ZB = 640
FB = 16


def _mesh():
    return plsc.VectorSubcoreMesh(
        core_axis_name="c", subcore_axis_name="s", num_cores=NC, num_subcores=NS
    )


def _sc_degree(dst_r):
    @functools.partial(
        pl.kernel,
        out_type=jax.ShapeDtypeStruct((NC * NPAD,), jnp.float32),
        mesh=_mesh(),
        scratch_types=[
            pltpu.VMEM((DEG_CH, AGG_W), jnp.int32),
            pltpu.VMEM((AGG_W,), jnp.float32),
            pltpu.VMEM((ZB,), jnp.float32),
            pltpu.VMEM_SHARED((NPAD,), jnp.float32),
        ],
        compiler_params=pltpu.CompilerParams(use_tc_tiling_on_sc=False),
    )
    def deg_kernel(dst_ref, out_ref, idxv, ones, zb, acc):
        c = lax.axis_index("c")
        s = lax.axis_index("s")
        one16 = jnp.ones((16,), jnp.float32)
        for j in range(AGG_W // 16):
            ones[pl.ds(j * 16, 16)] = one16
        z16 = jnp.zeros((16,), jnp.float32)
        for j in range(ZB // 16):
            zb[pl.ds(j * 16, 16)] = z16
        base = s * DROWS
        nzb = DROWS // ZB
        ztail = DROWS - nzb * ZB

        def zcopy(i, _):
            pltpu.sync_copy(zb, acc.at[pl.ds(base + i * ZB, ZB)])
            return 0

        lax.fori_loop(0, nzb, zcopy, 0)
        pltpu.sync_copy(zb.at[pl.ds(0, ztail)], acc.at[pl.ds(base + nzb * ZB, ztail)])
        plsc.subcore_barrier()

        wbase = c * DEG_NWC

        def body(g, _):
            pltpu.sync_copy(dst_ref.at[s, pl.ds(wbase + g * DEG_CH, DEG_CH)], idxv)
            for k in range(DEG_CH):
                pltpu.sync_copy(ones, acc.at[idxv.at[k]], add=True)
            return 0

        lax.fori_loop(0, DEG_NWC // DEG_CH, body, 0)
        plsc.subcore_barrier()

        def wb(i, _):
            pltpu.sync_copy(acc.at[pl.ds(base + i * ZB, ZB)], zb)
            pltpu.sync_copy(zb, out_ref.at[pl.ds(c * NPAD + base + i * ZB, ZB)])
            return 0

        lax.fori_loop(0, nzb, wb, 0)
        pltpu.sync_copy(acc.at[pl.ds(base + nzb * ZB, ztail)], zb.at[pl.ds(0, ztail)])
        pltpu.sync_copy(
            zb.at[pl.ds(0, ztail)],
            out_ref.at[pl.ds(c * NPAD + base + nzb * ZB, ztail)],
        )

    return deg_kernel(dst_r)


def _sc_aggregate(hs_flat, comb_r, bpc):
    @functools.partial(
        pl.kernel,
        out_type=jax.ShapeDtypeStruct((NC * bpc, NPAD, FB), jnp.float32),
        mesh=_mesh(),
        scratch_types=[
            pltpu.VMEM((2, SUP, AGG_W), jnp.int32),
            pltpu.VMEM((2, AGG_W, FB), jnp.float32),
            pltpu.VMEM((2, AGG_W, FB), jnp.float32),
            pltpu.VMEM_SHARED((NPAD, FB), jnp.float32),
            pltpu.SemaphoreType.DMA,
            pltpu.SemaphoreType.DMA,
            pltpu.SemaphoreType.DMA,
            pltpu.SemaphoreType.DMA,
        ],
        compiler_params=pltpu.CompilerParams(use_tc_tiling_on_sc=False),
    )
    def agg_kernel(
        hs_ref, comb_ref, out_ref,
        sd, rowsA, rowsB, acc, semA, semB, semSA, semSB,
    ):
        core = lax.axis_index("c")
        s = lax.axis_index("s")
        base = s * ROWS_PER_TILE
        nz = ROWS_PER_TILE // AGG_W
        tail = ROWS_PER_TILE - nz * AGG_W
        z16 = jnp.zeros((16,), jnp.float32)

        for b in range(bpc):
            off = (core * bpc + b) * N

            def fire(p, rows, sem):
                for k in (2 * p, 2 * p + 1):
                    pltpu.async_copy(hs_ref.at[sd.at[0, k]], rows.at[k - 2 * p], sem)

            def wait_g(p, rows, sem):
                for k in (2 * p, 2 * p + 1):
                    pltpu.make_async_copy(
                        hs_ref.at[sd.at[0, k]], rows.at[k - 2 * p], sem
                    ).wait()

            def scat_fire(p, rows, semS):
                for k in (2 * p, 2 * p + 1):
                    pltpu.async_copy(rows.at[k - 2 * p], acc.at[sd.at[1, k]], semS, add=True)

            def scat_wait(p, rows, semS):
                for k in (2 * p, 2 * p + 1):
                    pltpu.make_async_copy(
                        rows.at[k - 2 * p], acc.at[sd.at[1, k]], semS
                    ).wait()

            def zfill(i, _):
                rowsA[0, i, pl.ds(0, 16)] = z16
                return 0

            lax.fori_loop(0, AGG_W, zfill, 0)

            def zcopy(i, _):
                pltpu.sync_copy(rowsA.at[0], acc.at[pl.ds(base + i * AGG_W, AGG_W)])
                return 0

            lax.fori_loop(0, nz, zcopy, 0)
            pltpu.sync_copy(
                rowsA.at[0, pl.ds(0, tail)], acc.at[pl.ds(base + nz * AGG_W, tail)]
            )
            plsc.subcore_barrier()

            def superchunk(g, _):
                pltpu.sync_copy(comb_ref.at[s, g], sd)
                for k in range(SUP):
                    for j in range(AGG_W // 16):
                        sd[0, k, pl.ds(j * 16, 16)] = sd[0, k, pl.ds(j * 16, 16)] + off
                fire(0, rowsA, semA)
                fire(1, rowsB, semB)
                wait_g(0, rowsA, semA)
                scat_fire(0, rowsA, semSA)
                wait_g(1, rowsB, semB)
                scat_fire(1, rowsB, semSB)
                scat_wait(0, rowsA, semSA)
                fire(2, rowsA, semA)
                scat_wait(1, rowsB, semSB)
                fire(3, rowsB, semB)
                wait_g(2, rowsA, semA)
                scat_fire(2, rowsA, semSA)
                wait_g(3, rowsB, semB)
                scat_fire(3, rowsB, semSB)
                scat_wait(2, rowsA, semSA)
                scat_wait(3, rowsB, semSB)
                return 0

            lax.fori_loop(0, NSUP, superchunk, 0)

            plsc.subcore_barrier()

            for cblk in range(NC):
                blk = cblk * bpc + b

                @pl.when(core == cblk)
                def _():
                    def wb(i, _):
                        pltpu.sync_copy(acc.at[pl.ds(base + i * AGG_W, AGG_W)], rowsA.at[0])
                        pltpu.sync_copy(
                            rowsA.at[0], out_ref.at[blk, pl.ds(base + i * AGG_W, AGG_W)]
                        )
                        return 0

                    lax.fori_loop(0, nz, wb, 0)
                    pltpu.sync_copy(
                        acc.at[pl.ds(base + nz * AGG_W, tail)], rowsA.at[0, pl.ds(0, tail)]
                    )
                    pltpu.sync_copy(
                        rowsA.at[0, pl.ds(0, tail)],
                        out_ref.at[blk, pl.ds(base + nz * AGG_W, tail)],
                    )

    return agg_kernel(hs_flat, comb_r)


TN = 2000
GRID = N // TN


def _tc_layer1(xp, w1p, degp):
    def body(x_ref, w_ref, d_ref, hs_ref, dinv_ref):
        h = jnp.dot(x_ref[...], w_ref[...], preferred_element_type=jnp.float32)
        p = d_ref[...]
        dinv = lax.rsqrt(p[0] + p[1] + 1.0)
        hs = h * dinv
        for j in range(4):
            hs_ref[j] = hs[:, FB * j : FB * (j + 1)]
        dinv_ref[...] = dinv

    return pl.pallas_call(
        body,
        grid=(GRID,),
        in_specs=[
            pl.BlockSpec((TN, 64), lambda i: (i, 0)),
            pl.BlockSpec((64, 64), lambda i: (0, 0)),
            pl.BlockSpec((2, TN, 1), lambda i: (0, i, 0)),
        ],
        out_specs=[
            pl.BlockSpec((4, TN, FB), lambda i: (0, i, 0)),
            pl.BlockSpec((TN, 1), lambda i: (i, 0)),
        ],
        out_shape=[
            jax.ShapeDtypeStruct((4, N, FB), jnp.float32),
            jax.ShapeDtypeStruct((N, 1), jnp.float32),
        ],
    )(xp, w1p, degp)


def _tc_layer2(agg1, hs1, dinv, b1r, w2):
    def body(a_ref, h_ref, d_ref, b_ref, w_ref, out_ref):
        dinv = d_ref[...]
        u = (a_ref[...] + h_ref[...]) * dinv[None] + b_ref[...][:, None, :]
        t = jnp.maximum(u, 0.0)
        t64 = jnp.concatenate([t[j] for j in range(4)], axis=1)
        h2 = jnp.dot(t64, w_ref[...], preferred_element_type=jnp.float32)
        hs2 = h2 * dinv
        for j in range(8):
            out_ref[j] = hs2[:, FB * j : FB * (j + 1)]

    return pl.pallas_call(
        body,
        grid=(GRID,),
        in_specs=[
            pl.BlockSpec((4, TN, FB), lambda i: (0, i, 0)),
            pl.BlockSpec((4, TN, FB), lambda i: (0, i, 0)),
            pl.BlockSpec((TN, 1), lambda i: (i, 0)),
            pl.BlockSpec((4, FB), lambda i: (0, 0)),
            pl.BlockSpec((64, 128), lambda i: (0, 0)),
        ],
        out_specs=pl.BlockSpec((8, TN, FB), lambda i: (0, i, 0)),
        out_shape=jax.ShapeDtypeStruct((8, N, FB), jnp.float32),
    )(agg1, hs1, dinv, b1r, w2)


def _tc_final(agg2, hs2, dinv, b2r):
    def body(a_ref, h_ref, d_ref, b_ref, out_ref):
        u = (a_ref[...] + h_ref[...]) * d_ref[...][None] + b_ref[...][:, None, :]
        for j in range(8):
            out_ref[:, FB * j : FB * (j + 1)] = u[j]

    return pl.pallas_call(
        body,
        grid=(GRID,),
        in_specs=[
            pl.BlockSpec((8, TN, FB), lambda i: (0, i, 0)),
            pl.BlockSpec((8, TN, FB), lambda i: (0, i, 0)),
            pl.BlockSpec((TN, 1), lambda i: (i, 0)),
            pl.BlockSpec((8, FB), lambda i: (0, 0)),
        ],
        out_specs=pl.BlockSpec((TN, 128), lambda i: (i, 0)),
        out_shape=jax.ShapeDtypeStruct((N, 128), jnp.float32),
    )(agg2, hs2, dinv, b2r)


def kernel(x, edge_index, W1, b1, W2, b2):
    src = edge_index[0].astype(jnp.int32)
    dst = edge_index[1].astype(jnp.int32)
    npad = EPAD - E
    ar = jnp.arange(npad, dtype=jnp.int32)
    src_r = jnp.concatenate([src, (ar * 997) % N]).reshape(NS, AGG_NW, AGG_W)
    dst_r = jnp.concatenate([dst, N + (ar % (NPAD - N))]).reshape(NS, AGG_NW, AGG_W)
    comb_r = jnp.stack(
        [src_r.reshape(NS, NSUP, SUP, AGG_W), dst_r.reshape(NS, NSUP, SUP, AGG_W)],
        axis=2,
    )
    degp = _sc_degree(dst_r)
    degp3 = jnp.stack([degp[:N], degp[NPAD : NPAD + N]])[:, :, None]

    xp = jnp.pad(x, ((0, 0), (0, 64 - x.shape[1])))
    w1p = jnp.pad(W1, ((0, 64 - W1.shape[0]), (0, 0)))
    hs1, dinv = _tc_layer1(xp, w1p, degp3)

    agg1 = _sc_aggregate(hs1.reshape(4 * N, FB), comb_r, bpc=2)

    hs2 = _tc_layer2(agg1, hs1, dinv, b1.reshape(4, FB), W2)

    agg2 = _sc_aggregate(hs2.reshape(8 * N, FB), comb_r, bpc=4)

    return _tc_final(agg2, hs2, dinv, b2.reshape(8, FB))
